# baseline (speedup 1.0000x reference)
import jax
import jax.numpy as jnp
from jax.experimental import pallas as pl
from jax.experimental.pallas import tpu as pltpu

_N = 33554432
_CH = 1 << 20
_NCHUNKS = _N // _CH
_D = 4
_NBUF = 2 * _D


def _copy_body(x_ref, o_ref, buf, sem_in, sem_out):
    def in_cp(i):
        return pltpu.make_async_copy(
            x_ref.at[pl.ds(i * _CH, _CH)],
            buf.at[i % _NBUF],
            sem_in.at[i % _NBUF],
        )

    def out_cp(i):
        return pltpu.make_async_copy(
            buf.at[i % _NBUF],
            o_ref.at[pl.ds(i * _CH, _CH)],
            sem_out.at[i % _NBUF],
        )

    for i in range(_D):
        in_cp(i).start()
    for i in range(_NCHUNKS):
        in_cp(i).wait()
        if i == 0:
            idx = jax.lax.broadcasted_iota(jnp.int32, (128,), 0)
            buf[0, 0:128] = jnp.where(idx == 0, 0.0, buf[0, 0:128])
        out_cp(i).start()
        j = i + _D
        if j < _NCHUNKS:
            if j >= _NBUF:
                out_cp(j - _NBUF).wait()
            in_cp(j).start()
    for i in range(_NCHUNKS - _NBUF, _NCHUNKS):
        out_cp(i).wait()


def kernel(x):
    return pl.pallas_call(
        _copy_body,
        out_shape=jax.ShapeDtypeStruct((_N,), x.dtype),
        in_specs=[pl.BlockSpec(memory_space=pltpu.MemorySpace.HBM)],
        out_specs=pl.BlockSpec(memory_space=pltpu.MemorySpace.HBM),
        scratch_shapes=[
            pltpu.VMEM((_NBUF, _CH), jnp.float32),
            pltpu.SemaphoreType.DMA((_NBUF,)),
            pltpu.SemaphoreType.DMA((_NBUF,)),
        ],
    )(x)


# speedup vs baseline: 1.0017x; 1.0017x over previous
import jax
import jax.numpy as jnp
from jax.experimental import pallas as pl
from jax.experimental.pallas import tpu as pltpu

_N = 33554432
_CH = 1 << 19
_NCHUNKS = _N // _CH
_D = 8
_NBUF = 2 * _D


def _copy_body(x_ref, o_ref, buf, sem_in, sem_out):
    def in_cp(i):
        return pltpu.make_async_copy(
            x_ref.at[pl.ds(i * _CH, _CH)],
            buf.at[i % _NBUF],
            sem_in.at[i % _NBUF],
        )

    def out_cp(i):
        return pltpu.make_async_copy(
            buf.at[i % _NBUF],
            o_ref.at[pl.ds(i * _CH, _CH)],
            sem_out.at[i % _NBUF],
        )

    for i in range(_D):
        in_cp(i).start()
    for i in range(_NCHUNKS):
        in_cp(i).wait()
        if i == 0:
            idx = jax.lax.broadcasted_iota(jnp.int32, (128,), 0)
            buf[0, 0:128] = jnp.where(idx == 0, 0.0, buf[0, 0:128])
        out_cp(i).start()
        j = i + _D
        if j < _NCHUNKS:
            if j >= _NBUF:
                out_cp(j - _NBUF).wait()
            in_cp(j).start()
    for i in range(_NCHUNKS - _NBUF, _NCHUNKS):
        out_cp(i).wait()


def kernel(x):
    return pl.pallas_call(
        _copy_body,
        out_shape=jax.ShapeDtypeStruct((_N,), x.dtype),
        in_specs=[pl.BlockSpec(memory_space=pltpu.MemorySpace.HBM)],
        out_specs=pl.BlockSpec(memory_space=pltpu.MemorySpace.HBM),
        scratch_shapes=[
            pltpu.VMEM((_NBUF, _CH), jnp.float32),
            pltpu.SemaphoreType.DMA((_NBUF,)),
            pltpu.SemaphoreType.DMA((_NBUF,)),
        ],
    )(x)
